# baseline (device time: 12396 ns/iter reference)
import jax
import jax.numpy as jnp
from jax import lax
from jax.experimental import pallas as pl
from jax.experimental.pallas import tpu as pltpu

N_DEV = 4
M = 512
N = 512
Q = M // N_DEV
NCHUNK = 4
CR = Q // NCHUNK


def kernel(x):
    def body(x_ref, out_ref, sbuf, rbuf, s1, r1, s2, r2):
        my = lax.axis_index("i")

        barrier = pltpu.get_barrier_semaphore()
        for o in range(1, N_DEV):
            pl.semaphore_signal(
                barrier, inc=1,
                device_id=(lax.rem(my + o, N_DEV),),
                device_id_type=pl.DeviceIdType.MESH,
            )
        sbuf[...] = x_ref[0].astype(jnp.bfloat16)

        drain = []

        for c in range(NCHUNK):
            for o in range(1, N_DEV):
                tgt = lax.rem(my + o, N_DEV)
                rdma = pltpu.make_async_remote_copy(
                    src_ref=sbuf.at[pl.ds(tgt * Q + c * CR, CR)],
                    dst_ref=rbuf.at[N_DEV - 1 - o, pl.ds(c * CR, CR)],
                    send_sem=s1.at[c, o - 1],
                    recv_sem=r1.at[c, N_DEV - 1 - o],
                    device_id=(tgt,),
                    device_id_type=pl.DeviceIdType.MESH,
                )
                rdma.start()
                drain.append(rdma)

        for c in range(NCHUNK):
            acc = x_ref[0, pl.ds(my * Q + c * CR, CR), :]
            for s in (0, 2, 1):
                recv = pltpu.make_async_remote_copy(
                    src_ref=sbuf.at[pl.ds(0, CR)],
                    dst_ref=rbuf.at[s, pl.ds(c * CR, CR)],
                    send_sem=s1.at[c, 0],
                    recv_sem=r1.at[c, s],
                    device_id=(my,),
                    device_id_type=pl.DeviceIdType.MESH,
                )
                recv.wait_recv()
                acc = acc + rbuf[s, pl.ds(c * CR, CR), :].astype(jnp.float32)
            out_ref[pl.ds(my * Q + c * CR, CR), :] = acc.astype(jnp.bfloat16)

            for o in range(1, N_DEV):
                tgt = lax.rem(my + o, N_DEV)
                rdma = pltpu.make_async_remote_copy(
                    src_ref=out_ref.at[pl.ds(my * Q + c * CR, CR)],
                    dst_ref=out_ref.at[pl.ds(my * Q + c * CR, CR)],
                    send_sem=s2.at[c, o - 1],
                    recv_sem=r2.at[c, N_DEV - 1 - o],
                    device_id=(tgt,),
                    device_id_type=pl.DeviceIdType.MESH,
                )
                rdma.start()
                drain.append(rdma)

        for c in range(NCHUNK):
            for s in range(N_DEV - 1):
                src_dev = lax.rem(my + s + 1, N_DEV)
                recv = pltpu.make_async_remote_copy(
                    src_ref=sbuf.at[pl.ds(0, CR)],
                    dst_ref=out_ref.at[pl.ds(src_dev * Q + c * CR, CR)],
                    send_sem=s2.at[c, 0],
                    recv_sem=r2.at[c, s],
                    device_id=(my,),
                    device_id_type=pl.DeviceIdType.MESH,
                )
                recv.wait_recv()

        pl.semaphore_wait(barrier, N_DEV - 1)
        for rdma in drain:
            rdma.wait_send()

    return pl.pallas_call(
        body,
        out_shape=jax.ShapeDtypeStruct((M, N), jnp.bfloat16),
        in_specs=[pl.BlockSpec(memory_space=pltpu.VMEM)],
        out_specs=pl.BlockSpec(memory_space=pltpu.VMEM),
        scratch_shapes=[
            pltpu.VMEM((M, N), jnp.bfloat16),
            pltpu.VMEM((N_DEV - 1, Q, N), jnp.bfloat16),
            pltpu.SemaphoreType.DMA((NCHUNK, N_DEV - 1)),
            pltpu.SemaphoreType.DMA((NCHUNK, N_DEV - 1)),
            pltpu.SemaphoreType.DMA((NCHUNK, N_DEV - 1)),
            pltpu.SemaphoreType.DMA((NCHUNK, N_DEV - 1)),
        ],
        compiler_params=pltpu.CompilerParams(collective_id=0),
    )(x)


# device time: 12342 ns/iter; 1.0044x vs baseline; 1.0044x over previous
import jax
import jax.numpy as jnp
from jax import lax
from jax.experimental import pallas as pl
from jax.experimental.pallas import tpu as pltpu

N_DEV = 4
M = 512
N = 512
Q = M // N_DEV
NCHUNK = 4
CR = Q // NCHUNK


def kernel(x):
    def body(x_ref, out_ref, sbuf, rbuf, s1, r1, s2, r2):
        my = lax.axis_index("i")

        barrier = pltpu.get_barrier_semaphore()
        for o in range(1, N_DEV):
            pl.semaphore_signal(
                barrier, inc=1,
                device_id=(lax.rem(my + o, N_DEV),),
                device_id_type=pl.DeviceIdType.MESH,
            )
        sbuf[...] = x_ref[0].astype(jnp.bfloat16)

        drain = []

        for c in range(NCHUNK):
            for o in (2, 1, 3):
                tgt = lax.rem(my + o, N_DEV)
                rdma = pltpu.make_async_remote_copy(
                    src_ref=sbuf.at[pl.ds(tgt * Q + c * CR, CR)],
                    dst_ref=rbuf.at[N_DEV - 1 - o, pl.ds(c * CR, CR)],
                    send_sem=s1.at[c, o - 1],
                    recv_sem=r1.at[c, N_DEV - 1 - o],
                    device_id=(tgt,),
                    device_id_type=pl.DeviceIdType.MESH,
                )
                rdma.start()
                drain.append(rdma)

        for c in range(NCHUNK):
            acc = x_ref[0, pl.ds(my * Q + c * CR, CR), :]
            for s in (0, 2, 1):
                recv = pltpu.make_async_remote_copy(
                    src_ref=sbuf.at[pl.ds(0, CR)],
                    dst_ref=rbuf.at[s, pl.ds(c * CR, CR)],
                    send_sem=s1.at[c, 0],
                    recv_sem=r1.at[c, s],
                    device_id=(my,),
                    device_id_type=pl.DeviceIdType.MESH,
                )
                recv.wait_recv()
                acc = acc + rbuf[s, pl.ds(c * CR, CR), :].astype(jnp.float32)
            out_ref[pl.ds(my * Q + c * CR, CR), :] = acc.astype(jnp.bfloat16)

            for o in (2, 1, 3):
                tgt = lax.rem(my + o, N_DEV)
                rdma = pltpu.make_async_remote_copy(
                    src_ref=out_ref.at[pl.ds(my * Q + c * CR, CR)],
                    dst_ref=out_ref.at[pl.ds(my * Q + c * CR, CR)],
                    send_sem=s2.at[c, o - 1],
                    recv_sem=r2.at[c, N_DEV - 1 - o],
                    device_id=(tgt,),
                    device_id_type=pl.DeviceIdType.MESH,
                )
                rdma.start()
                drain.append(rdma)

        for c in range(NCHUNK):
            for s in range(N_DEV - 1):
                src_dev = lax.rem(my + s + 1, N_DEV)
                recv = pltpu.make_async_remote_copy(
                    src_ref=sbuf.at[pl.ds(0, CR)],
                    dst_ref=out_ref.at[pl.ds(src_dev * Q + c * CR, CR)],
                    send_sem=s2.at[c, 0],
                    recv_sem=r2.at[c, s],
                    device_id=(my,),
                    device_id_type=pl.DeviceIdType.MESH,
                )
                recv.wait_recv()

        pl.semaphore_wait(barrier, N_DEV - 1)
        for rdma in drain:
            rdma.wait_send()

    return pl.pallas_call(
        body,
        out_shape=jax.ShapeDtypeStruct((M, N), jnp.bfloat16),
        in_specs=[pl.BlockSpec(memory_space=pltpu.VMEM)],
        out_specs=pl.BlockSpec(memory_space=pltpu.VMEM),
        scratch_shapes=[
            pltpu.VMEM((M, N), jnp.bfloat16),
            pltpu.VMEM((N_DEV - 1, Q, N), jnp.bfloat16),
            pltpu.SemaphoreType.DMA((NCHUNK, N_DEV - 1)),
            pltpu.SemaphoreType.DMA((NCHUNK, N_DEV - 1)),
            pltpu.SemaphoreType.DMA((NCHUNK, N_DEV - 1)),
            pltpu.SemaphoreType.DMA((NCHUNK, N_DEV - 1)),
        ],
        compiler_params=pltpu.CompilerParams(collective_id=0),
    )(x)
